# baseline (speedup 1.0000x reference)
import jax
import jax.numpy as jnp
from jax.experimental import pallas as pl
from jax.experimental.pallas import tpu as pltpu

_HF = 112
_CBLK = 16
_C0 = 240


def _rot_body(y_ref, x_ref, o_ref):
    del y_ref
    j = pl.program_id(1)
    nblk = 48 // _CBLK
    g = 1 + j // nblk

    @pl.when(g == 1)
    def _():
        o_ref[:, :, :_HF, :_HF] = x_ref[:, :, :_HF, _HF:]
        o_ref[:, :, :_HF, _HF:] = x_ref[:, :, _HF:, :_HF]
        o_ref[:, :, _HF:, :_HF] = x_ref[:, :, _HF:, _HF:]
        o_ref[:, :, _HF:, _HF:] = x_ref[:, :, :_HF, :_HF]

    @pl.when(g == 2)
    def _():
        o_ref[:, :, :_HF, :] = x_ref[:, :, _HF:, :]
        o_ref[:, :, _HF:, :] = x_ref[:, :, :_HF, :]

    @pl.when(g == 3)
    def _():
        o_ref[:, :, :_HF, :_HF] = x_ref[:, :, _HF:, _HF:]
        o_ref[:, :, :_HF, _HF:] = x_ref[:, :, :_HF, :_HF]
        o_ref[:, :, _HF:, :_HF] = x_ref[:, :, :_HF, _HF:]
        o_ref[:, :, _HF:, _HF:] = x_ref[:, :, _HF:, :_HF]


def kernel(x):
    b, c, h, w = x.shape
    y = x + 0.0
    blk = pl.BlockSpec((1, _CBLK, h, w), lambda i, j: (i, _C0 // _CBLK + j, 0, 0))
    return pl.pallas_call(
        _rot_body,
        grid=(b, (c - _C0) // _CBLK),
        in_specs=[pl.BlockSpec(memory_space=pl.ANY), blk],
        out_specs=blk,
        out_shape=jax.ShapeDtypeStruct(x.shape, x.dtype),
        input_output_aliases={0: 0},
        compiler_params=pltpu.CompilerParams(
            dimension_semantics=("parallel", "parallel"),
        ),
    )(y, x)
